# Initial kernel scaffold; baseline (speedup 1.0000x reference)
import jax
import jax.numpy as jnp
from jax.experimental import pallas as pl


def kernel(x, edge_index, W, b):
    raise NotImplementedError("write your pallas kernel here")


# speedup vs baseline: 6.3963x; 6.3963x over previous
import functools

import jax
import jax.numpy as jnp
from jax import lax
from jax.experimental import pallas as pl
from jax.experimental.pallas import tpu as pltpu
from jax.experimental.pallas import tpu_sc as plsc

N = 10000
E = 320000
D = 128
NC = 2
NS = 16
B = 128

NPAD = 10240
EPAD = 327680
EW = EPAD // (NC * NS)
NCH = EW // B
EC = EPAD // NS
NCH_DEG = EC // B
ROWS_PER_TILE = NPAD // NS

_MESH = plsc.VectorSubcoreMesh(
    core_axis_name="c", subcore_axis_name="s", num_cores=NC, num_subcores=NS
)


def _zero_vmem_2d(ref, nrows, ncols):
    z = jnp.zeros((16,), jnp.float32)

    def row(r, _):
        for k in range(ncols // 16):
            ref[r, pl.ds(k * 16, 16)] = z
        return _

    lax.fori_loop(0, nrows, row, None)


@functools.partial(
    pl.kernel,
    out_type=jax.ShapeDtypeStruct((NC, NPAD), jnp.float32),
    mesh=_MESH,
    scratch_types=[
        pltpu.VMEM((B,), jnp.int32),
        pltpu.VMEM((B,), jnp.float32),
        pltpu.VMEM((ROWS_PER_TILE,), jnp.float32),
        pltpu.VMEM_SHARED((NPAD,), jnp.float32),
    ],
)
def _degree_kernel(edges_hbm, deg_hbm, idx_v, ones_v, zrow_v, deg_acc):
    c = lax.axis_index("c")
    s = lax.axis_index("s")
    one = jnp.ones((16,), jnp.float32)
    zero = jnp.zeros((16,), jnp.float32)
    for k in range(B // 16):
        ones_v[pl.ds(k * 16, 16)] = one
    for k in range(ROWS_PER_TILE // 16):
        zrow_v[pl.ds(k * 16, 16)] = zero
    pltpu.sync_copy(zrow_v, deg_acc.at[pl.ds(s * ROWS_PER_TILE, ROWS_PER_TILE)])
    plsc.subcore_barrier()

    def chunk(j, _):
        base = s * EC + j * B
        pltpu.sync_copy(edges_hbm.at[c, pl.ds(base, B)], idx_v)
        pltpu.sync_copy(ones_v, deg_acc.at[idx_v], add=True)
        return _

    lax.fori_loop(0, NCH_DEG, chunk, None)
    plsc.subcore_barrier()
    pltpu.sync_copy(
        deg_acc.at[pl.ds(s * ROWS_PER_TILE, ROWS_PER_TILE)],
        deg_hbm.at[c, pl.ds(s * ROWS_PER_TILE, ROWS_PER_TILE)],
    )


def _matmul_body(deg_ref, x_ref, w_ref, o_ref):
    norm = lax.rsqrt(jnp.maximum(deg_ref[...], 1.0))
    o_ref[...] = jnp.dot(
        x_ref[...] * norm, w_ref[...], preferred_element_type=jnp.float32
    )


def _scaled_matmul(deg_col, xp, w):
    return pl.pallas_call(
        _matmul_body,
        grid=(NPAD // 128,),
        in_specs=[
            pl.BlockSpec((128, 1), lambda i: (i, 0)),
            pl.BlockSpec((128, D), lambda i: (i, 0)),
            pl.BlockSpec((D, D), lambda i: (0, 0)),
        ],
        out_specs=pl.BlockSpec((128, D), lambda i: (i, 0)),
        out_shape=jax.ShapeDtypeStruct((NPAD, D), jnp.float32),
    )(deg_col, xp, w)


@functools.partial(
    pl.kernel,
    out_type=jax.ShapeDtypeStruct((NC, NPAD, D), jnp.float32),
    mesh=_MESH,
    scratch_types=[
        pltpu.VMEM((B,), jnp.int32),
        pltpu.VMEM((B,), jnp.int32),
        pltpu.VMEM((B,), jnp.int32),
        pltpu.VMEM((B,), jnp.int32),
        pltpu.VMEM((B, D), jnp.float32),
        pltpu.VMEM((B, D), jnp.float32),
        pltpu.VMEM((64, D), jnp.float32),
        pltpu.VMEM_SHARED((NPAD, D), jnp.float32),
        pltpu.SemaphoreType.DMA,
        pltpu.SemaphoreType.DMA,
    ],
)
def _message_kernel(
    h_hbm, srcp_hbm, dstp_hbm, out_hbm,
    s0, d0, s1, d1, rows0, rows1, zbuf, acc, sem0, sem1,
):
    c = lax.axis_index("c")
    s = lax.axis_index("s")
    wid = s * NC + c
    base = wid * EW

    _zero_vmem_2d(zbuf, 64, D)

    def zrow(k, _):
        pltpu.sync_copy(zbuf, acc.at[pl.ds(s * ROWS_PER_TILE + k * 64, 64)])
        return _

    lax.fori_loop(0, ROWS_PER_TILE // 64, zrow, None)
    plsc.subcore_barrier()

    pltpu.sync_copy(srcp_hbm.at[pl.ds(base, B)], s0)
    pltpu.sync_copy(dstp_hbm.at[pl.ds(base, B)], d0)
    pltpu.async_copy(h_hbm.at[s0], rows0, sem0)

    def pair(jj, _):
        b1 = base + (2 * jj + 1) * B
        pltpu.sync_copy(srcp_hbm.at[pl.ds(b1, B)], s1)
        pltpu.sync_copy(dstp_hbm.at[pl.ds(b1, B)], d1)
        pltpu.async_copy(h_hbm.at[s1], rows1, sem1)
        pltpu.make_async_copy(h_hbm.at[s0], rows0, sem0).wait()
        pltpu.sync_copy(rows0, acc.at[d0], add=True)

        @pl.when(2 * jj + 2 < NCH)
        def _prefetch():
            b2 = base + (2 * jj + 2) * B
            pltpu.sync_copy(srcp_hbm.at[pl.ds(b2, B)], s0)
            pltpu.sync_copy(dstp_hbm.at[pl.ds(b2, B)], d0)
            pltpu.async_copy(h_hbm.at[s0], rows0, sem0)

        pltpu.make_async_copy(h_hbm.at[s1], rows1, sem1).wait()
        pltpu.sync_copy(rows1, acc.at[d1], add=True)
        return _

    lax.fori_loop(0, NCH // 2, pair, None)
    plsc.subcore_barrier()
    pltpu.sync_copy(
        acc.at[pl.ds(s * ROWS_PER_TILE, ROWS_PER_TILE)],
        out_hbm.at[c, pl.ds(s * ROWS_PER_TILE, ROWS_PER_TILE)],
    )


def _finalize_body(deg_ref, p0_ref, p1_ref, b_ref, o_ref):
    norm = lax.rsqrt(jnp.maximum(deg_ref[...], 1.0))
    o_ref[...] = (p0_ref[...] + p1_ref[...]) * norm + b_ref[...]


def _finalize(deg_col, p0, p1, bias_row):
    return pl.pallas_call(
        _finalize_body,
        grid=(NPAD // 128,),
        in_specs=[
            pl.BlockSpec((128, 1), lambda i: (i, 0)),
            pl.BlockSpec((128, D), lambda i: (i, 0)),
            pl.BlockSpec((128, D), lambda i: (i, 0)),
            pl.BlockSpec((1, D), lambda i: (0, 0)),
        ],
        out_specs=pl.BlockSpec((128, D), lambda i: (i, 0)),
        out_shape=jax.ShapeDtypeStruct((NPAD, D), jnp.float32),
    )(deg_col, p0, p1, bias_row)


def kernel(x, edge_index, W, b):
    src = edge_index[0].astype(jnp.int32)
    dst = edge_index[1].astype(jnp.int32)
    pad = N + (jnp.arange(EPAD - E, dtype=jnp.int32) % (NPAD - N))
    srcp = jnp.concatenate([src, pad])
    dstp = jnp.concatenate([dst, pad])
    edges_p = jnp.stack([srcp, dstp])
    xp = jnp.zeros((NPAD, D), jnp.float32).at[:N].set(x)

    degs = _degree_kernel(edges_p)
    h = _scaled_matmul(degs[0].reshape(NPAD, 1), xp, W)
    parts = _message_kernel(h, srcp, dstp)
    out = _finalize(
        degs[1].reshape(NPAD, 1), parts[0], parts[1], b.reshape(1, D)
    )
    return out[:N]


# speedup vs baseline: 10.7981x; 1.6882x over previous
import functools

import jax
import jax.numpy as jnp
from jax import lax
from jax.experimental import pallas as pl
from jax.experimental.pallas import tpu as pltpu
from jax.experimental.pallas import tpu_sc as plsc

N = 10000
E = 320000
D = 128
NC = 2
NS = 16
B = 128

EW = E // (NC * NS)
NCH = 78
TAIL = EW - NCH * B
RING = 2
ROWS_PER_TILE = N // NS
DEG_SLICE = 640
NDEG = 10240
ACC_SLICE = 1000

_MESH = plsc.VectorSubcoreMesh(
    core_axis_name="c", subcore_axis_name="s", num_cores=NC, num_subcores=NS
)


@functools.partial(
    pl.kernel,
    out_type=jax.ShapeDtypeStruct((NC * NDEG,), jnp.float32),
    mesh=_MESH,
    scratch_types=[
        [pltpu.VMEM((B,), jnp.int32)] * 2,
        pltpu.VMEM((TAIL,), jnp.int32),
        pltpu.VMEM((B,), jnp.float32),
        pltpu.VMEM((DEG_SLICE,), jnp.float32),
        pltpu.VMEM_SHARED((NDEG,), jnp.float32),
        [pltpu.SemaphoreType.DMA] * 2,
        [pltpu.SemaphoreType.DMA] * 2,
    ],
)
def _degree_kernel(
    src_hbm, deg_hbm, idx, tidx, ones_v, zrow_v, deg_acc, lsems, ssems
):
    c = lax.axis_index("c")
    s = lax.axis_index("s")
    one = jnp.ones((16,), jnp.float32)
    zero = jnp.zeros((16,), jnp.float32)
    for k in range(B // 16):
        ones_v[pl.ds(k * 16, 16)] = one
    for k in range(DEG_SLICE // 16):
        zrow_v[pl.ds(k * 16, 16)] = zero

    pltpu.sync_copy(zrow_v, deg_acc.at[pl.ds(s * DEG_SLICE, DEG_SLICE)])
    plsc.subcore_barrier()

    base = (s * NC + c) * EW
    for b in range(2):
        pltpu.async_copy(
            src_hbm.at[pl.ds(base + b * B, B)], idx[b], lsems[b]
        )

    def pair(jj, _):
        j0 = 2 * jj
        for b in range(2):
            j = j0 + b
            pltpu.make_async_copy(
                src_hbm.at[pl.ds(base, B)], idx[b], lsems[b]
            ).wait()
            pltpu.async_copy(ones_v, deg_acc.at[idx[b]], ssems[b], add=True)

            @pl.when(j + 2 < NCH)
            def _next():
                pltpu.make_async_copy(
                    ones_v, deg_acc.at[idx[b]], ssems[b]
                ).wait()
                pltpu.async_copy(
                    src_hbm.at[pl.ds(base + (j + 2) * B, B)],
                    idx[b],
                    lsems[b],
                )

        return _

    lax.fori_loop(0, NCH // 2, pair, None)
    for b in range(2):
        pltpu.make_async_copy(ones_v, deg_acc.at[idx[b]], ssems[b]).wait()
    pltpu.sync_copy(src_hbm.at[pl.ds(base + NCH * B, TAIL)], tidx)
    pltpu.sync_copy(ones_v.at[pl.ds(0, TAIL)], deg_acc.at[tidx], add=True)
    plsc.subcore_barrier()

    pltpu.sync_copy(
        deg_acc.at[pl.ds(s * DEG_SLICE, DEG_SLICE)],
        deg_hbm.at[pl.ds(c * NDEG + s * DEG_SLICE, DEG_SLICE)],
    )


def _matmul_body(deg_ref, x_ref, w_ref, o_ref):
    norm = lax.rsqrt(jnp.maximum(deg_ref[0] + deg_ref[1], 1.0))
    o_ref[...] = jnp.dot(
        x_ref[...] * norm, w_ref[...], preferred_element_type=jnp.float32
    )


def _scaled_matmul(deg_col, x, w):
    return pl.pallas_call(
        _matmul_body,
        grid=(N // 2000,),
        in_specs=[
            pl.BlockSpec((NC, 2000, 1), lambda i: (0, i, 0)),
            pl.BlockSpec((2000, D), lambda i: (i, 0)),
            pl.BlockSpec((D, D), lambda i: (0, 0)),
        ],
        out_specs=pl.BlockSpec((2000, D), lambda i: (i, 0)),
        out_shape=jax.ShapeDtypeStruct((N, D), jnp.float32),
    )(deg_col, x, w)


@functools.partial(
    pl.kernel,
    out_type=(
        jax.ShapeDtypeStruct((NC * N, D), jnp.float32),
        jax.ShapeDtypeStruct((NC * NDEG,), jnp.float32),
    ),
    mesh=_MESH,
    scratch_types=[
        pltpu.VMEM((EW,), jnp.int32),
        [pltpu.VMEM((B,), jnp.int32)] * RING,
        pltpu.VMEM((TAIL,), jnp.int32),
        [pltpu.VMEM((B, D), jnp.float32)] * RING,
        pltpu.VMEM((TAIL, D), jnp.float32),
        pltpu.VMEM((8, D), jnp.float32),
        pltpu.VMEM((B,), jnp.float32),
        pltpu.VMEM((DEG_SLICE,), jnp.float32),
        pltpu.VMEM_SHARED((N, D), jnp.float32),
        pltpu.VMEM_SHARED((NDEG,), jnp.float32),
        [pltpu.SemaphoreType.DMA] * RING,
        [pltpu.SemaphoreType.DMA] * RING,
        [pltpu.SemaphoreType.DMA] * RING,
        [pltpu.SemaphoreType.DMA] * RING,
    ],
)
def _message_kernel(
    h_hbm, src2_hbm, dst_hbm, out_hbm, deg_hbm,
    sidx, didx, tidx, rows, trows, zbuf, ones_v, zrow_v, acc, deg_acc,
    gsems, ssems, lsems, dsems,
):
    c = lax.axis_index("c")
    s = lax.axis_index("s")
    wid = s * NC + c
    base = wid * EW

    one = jnp.ones((16,), jnp.float32)
    zero = jnp.zeros((16,), jnp.float32)
    for k in range(B // 16):
        ones_v[pl.ds(k * 16, 16)] = one
    for k in range(DEG_SLICE // 16):
        zrow_v[pl.ds(k * 16, 16)] = zero

    def zrow_body(r, _):
        for k in range(D // 16):
            zbuf[r, pl.ds(k * 16, 16)] = zero
        return _

    lax.fori_loop(0, 8, zrow_body, None)

    def zcopy(k, _):
        pltpu.sync_copy(zbuf, acc.at[pl.ds(s * ACC_SLICE + k * 8, 8)])
        return _

    @pl.when(s < N // ACC_SLICE)
    def _zero_acc():
        lax.fori_loop(0, ACC_SLICE // 8, zcopy, None)

    pltpu.sync_copy(zrow_v, deg_acc.at[pl.ds(s * DEG_SLICE, DEG_SLICE)])
    plsc.subcore_barrier()

    pltpu.sync_copy(src2_hbm.at[pl.ds(base, EW)], sidx)

    for b in range(RING):
        pltpu.async_copy(
            dst_hbm.at[pl.ds(base + b * B, B)], didx[b], lsems[b]
        )
        pltpu.async_copy(
            h_hbm.at[sidx.at[pl.ds(b * B, B)]], rows[b], gsems[b]
        )

    def super_iter(jj, _):
        j0 = jj * RING
        for b in range(RING):
            j = j0 + b
            pltpu.make_async_copy(
                h_hbm.at[sidx.at[pl.ds(0, B)]], rows[b], gsems[b]
            ).wait()
            pltpu.make_async_copy(
                dst_hbm.at[pl.ds(base, B)], didx[b], lsems[b]
            ).wait()
            pltpu.async_copy(rows[b], acc.at[didx[b]], ssems[b], add=True)
            pltpu.async_copy(ones_v, deg_acc.at[didx[b]], dsems[b], add=True)

            @pl.when(j + RING < NCH)
            def _next():
                pltpu.make_async_copy(
                    rows[b], acc.at[didx[b]], ssems[b]
                ).wait()
                pltpu.make_async_copy(
                    ones_v, deg_acc.at[didx[b]], dsems[b]
                ).wait()
                pltpu.async_copy(
                    dst_hbm.at[pl.ds(base + (j + RING) * B, B)],
                    didx[b],
                    lsems[b],
                )
                pltpu.async_copy(
                    h_hbm.at[sidx.at[pl.ds((j + RING) * B, B)]],
                    rows[b],
                    gsems[b],
                )

        return _

    lax.fori_loop(0, NCH // RING, super_iter, None)
    for b in range(RING):
        pltpu.make_async_copy(rows[b], acc.at[didx[b]], ssems[b]).wait()
        pltpu.make_async_copy(ones_v, deg_acc.at[didx[b]], dsems[b]).wait()

    pltpu.sync_copy(dst_hbm.at[pl.ds(base + NCH * B, TAIL)], tidx)
    pltpu.async_copy(
        h_hbm.at[sidx.at[pl.ds(NCH * B, TAIL)]], trows, gsems[0]
    ).wait()
    pltpu.sync_copy(trows, acc.at[tidx], add=True)
    pltpu.sync_copy(ones_v.at[pl.ds(0, TAIL)], deg_acc.at[tidx], add=True)

    plsc.subcore_barrier()

    @pl.when(s < N // ACC_SLICE)
    def _write_out():
        pltpu.sync_copy(
            acc.at[pl.ds(s * ACC_SLICE, ACC_SLICE)],
            out_hbm.at[pl.ds(c * N + s * ACC_SLICE, ACC_SLICE)],
        )

    pltpu.sync_copy(
        deg_acc.at[pl.ds(s * DEG_SLICE, DEG_SLICE)],
        deg_hbm.at[pl.ds(c * NDEG + s * DEG_SLICE, DEG_SLICE)],
    )


def _finalize_body(deg_ref, parts_ref, b_ref, o_ref):
    norm = lax.rsqrt(jnp.maximum(deg_ref[0] + deg_ref[1], 1.0))
    o_ref[...] = (parts_ref[0] + parts_ref[1]) * norm + b_ref[...]


def _finalize(deg_col, parts, bias_row):
    return pl.pallas_call(
        _finalize_body,
        grid=(N // 2000,),
        in_specs=[
            pl.BlockSpec((NC, 2000, 1), lambda i: (0, i, 0)),
            pl.BlockSpec((NC, 2000, D), lambda i: (0, i, 0)),
            pl.BlockSpec((1, D), lambda i: (0, 0)),
        ],
        out_specs=pl.BlockSpec((2000, D), lambda i: (i, 0)),
        out_shape=jax.ShapeDtypeStruct((N, D), jnp.float32),
    )(deg_col, parts, bias_row)


def kernel(x, edge_index, W, b):
    src = edge_index[0].astype(jnp.int32)
    dst = edge_index[1].astype(jnp.int32)
    deg_out_parts = _degree_kernel(src)
    h = _scaled_matmul(deg_out_parts.reshape(NC, NDEG, 1), x, W)
    parts, deg_in_parts = _message_kernel(h, src, dst)
    return _finalize(
        deg_in_parts.reshape(NC, NDEG, 1),
        parts.reshape(NC, N, D),
        b.reshape(1, D),
    )


# speedup vs baseline: 11.5169x; 1.0666x over previous
import functools

import jax
import jax.numpy as jnp
from jax import lax
from jax.experimental import pallas as pl
from jax.experimental.pallas import tpu as pltpu
from jax.experimental.pallas import tpu_sc as plsc

N = 10000
E = 320000
D = 128
NC = 2
NS = 16
B = 128

EW = E // (NC * NS)
NCH = 78
TAIL = EW - NCH * B
RING = 2
ROWS_PER_TILE = N // NS
DEG_SLICE = 640
NDEG = 10240
ACC_SLICE = 1000

_MESH = plsc.VectorSubcoreMesh(
    core_axis_name="c", subcore_axis_name="s", num_cores=NC, num_subcores=NS
)


@functools.partial(
    pl.kernel,
    out_type=jax.ShapeDtypeStruct((NC * NDEG,), jnp.float32),
    mesh=_MESH,
    scratch_types=[
        [pltpu.VMEM((B,), jnp.int32)] * 2,
        pltpu.VMEM((TAIL,), jnp.int32),
        pltpu.VMEM((B,), jnp.float32),
        pltpu.VMEM_SHARED((NDEG,), jnp.float32),
        [pltpu.SemaphoreType.DMA] * 2,
        [pltpu.SemaphoreType.DMA] * 2,
    ],
)
def _degree_kernel(
    src_hbm, z1_hbm, deg_hbm, idx, tidx, ones_v, deg_acc, lsems, ssems
):
    c = lax.axis_index("c")
    s = lax.axis_index("s")
    one = jnp.ones((16,), jnp.float32)
    for k in range(B // 16):
        ones_v[pl.ds(k * 16, 16)] = one
    pltpu.sync_copy(z1_hbm, deg_acc.at[pl.ds(s * DEG_SLICE, DEG_SLICE)])
    plsc.subcore_barrier()

    base = (s * NC + c) * EW
    for b in range(2):
        pltpu.async_copy(
            src_hbm.at[pl.ds(base + b * B, B)], idx[b], lsems[b]
        )

    def pair(jj, _):
        j0 = 2 * jj
        for b in range(2):
            j = j0 + b
            pltpu.make_async_copy(
                src_hbm.at[pl.ds(base, B)], idx[b], lsems[b]
            ).wait()
            pltpu.async_copy(ones_v, deg_acc.at[idx[b]], ssems[b], add=True)

            @pl.when(j + 2 < NCH)
            def _next():
                pltpu.make_async_copy(
                    ones_v, deg_acc.at[idx[b]], ssems[b]
                ).wait()
                pltpu.async_copy(
                    src_hbm.at[pl.ds(base + (j + 2) * B, B)],
                    idx[b],
                    lsems[b],
                )

        return _

    lax.fori_loop(0, NCH // 2, pair, None)
    for b in range(2):
        pltpu.make_async_copy(ones_v, deg_acc.at[idx[b]], ssems[b]).wait()
    pltpu.sync_copy(src_hbm.at[pl.ds(base + NCH * B, TAIL)], tidx)
    pltpu.sync_copy(ones_v.at[pl.ds(0, TAIL)], deg_acc.at[tidx], add=True)
    plsc.subcore_barrier()

    pltpu.sync_copy(
        deg_acc.at[pl.ds(s * DEG_SLICE, DEG_SLICE)],
        deg_hbm.at[pl.ds(c * NDEG + s * DEG_SLICE, DEG_SLICE)],
    )


def _matmul_body(deg_ref, x_ref, w_ref, o_ref):
    norm = lax.rsqrt(jnp.maximum(deg_ref[0] + deg_ref[1], 1.0))
    o_ref[...] = jnp.dot(
        x_ref[...] * norm, w_ref[...], preferred_element_type=jnp.float32
    )


def _scaled_matmul(deg_col, x, w):
    return pl.pallas_call(
        _matmul_body,
        grid=(N // 2000,),
        in_specs=[
            pl.BlockSpec((NC, 2000, 1), lambda i: (0, i, 0)),
            pl.BlockSpec((2000, D), lambda i: (i, 0)),
            pl.BlockSpec((D, D), lambda i: (0, 0)),
        ],
        out_specs=pl.BlockSpec((2000, D), lambda i: (i, 0)),
        out_shape=jax.ShapeDtypeStruct((N, D), jnp.float32),
    )(deg_col, x, w)


@functools.partial(
    pl.kernel,
    out_type=(
        jax.ShapeDtypeStruct((NC * NDEG, D), jnp.float32),
        jax.ShapeDtypeStruct((NC * NDEG,), jnp.float32),
    ),
    mesh=_MESH,
    scratch_types=[
        pltpu.VMEM((EW,), jnp.int32),
        [pltpu.VMEM((B,), jnp.int32)] * RING,
        pltpu.VMEM((TAIL,), jnp.int32),
        [pltpu.VMEM((B, D), jnp.float32)] * RING,
        pltpu.VMEM((TAIL, D), jnp.float32),
        pltpu.VMEM((B,), jnp.float32),
        pltpu.VMEM_SHARED((NDEG, D), jnp.float32),
        pltpu.VMEM_SHARED((NDEG,), jnp.float32),
        [pltpu.SemaphoreType.DMA] * RING,
        [pltpu.SemaphoreType.DMA] * RING,
        [pltpu.SemaphoreType.DMA] * RING,
        [pltpu.SemaphoreType.DMA] * RING,
    ],
)
def _message_kernel(
    h_hbm, src2_hbm, dst_hbm, z1_hbm, z2_hbm, out_hbm, deg_hbm,
    sidx, didx, tidx, rows, trows, ones_v, acc, deg_acc,
    gsems, ssems, lsems, dsems,
):
    c = lax.axis_index("c")
    s = lax.axis_index("s")
    wid = s * NC + c
    base = wid * EW

    one = jnp.ones((16,), jnp.float32)
    for k in range(B // 16):
        ones_v[pl.ds(k * 16, 16)] = one
    pltpu.sync_copy(z2_hbm, acc.at[pl.ds(s * DEG_SLICE, DEG_SLICE)])
    pltpu.sync_copy(z1_hbm, deg_acc.at[pl.ds(s * DEG_SLICE, DEG_SLICE)])
    plsc.subcore_barrier()

    pltpu.sync_copy(src2_hbm.at[pl.ds(base, EW)], sidx)

    for b in range(RING):
        pltpu.async_copy(
            dst_hbm.at[pl.ds(base + b * B, B)], didx[b], lsems[b]
        )
        pltpu.async_copy(
            h_hbm.at[sidx.at[pl.ds(b * B, B)]], rows[b], gsems[b]
        )

    def super_iter(jj, _):
        j0 = jj * RING
        for b in range(RING):
            j = j0 + b
            pltpu.make_async_copy(
                h_hbm.at[sidx.at[pl.ds(0, B)]], rows[b], gsems[b]
            ).wait()
            pltpu.make_async_copy(
                dst_hbm.at[pl.ds(base, B)], didx[b], lsems[b]
            ).wait()
            pltpu.async_copy(rows[b], acc.at[didx[b]], ssems[b], add=True)
            pltpu.async_copy(ones_v, deg_acc.at[didx[b]], dsems[b], add=True)

            @pl.when(j + RING < NCH)
            def _next():
                pltpu.make_async_copy(
                    rows[b], acc.at[didx[b]], ssems[b]
                ).wait()
                pltpu.make_async_copy(
                    ones_v, deg_acc.at[didx[b]], dsems[b]
                ).wait()
                pltpu.async_copy(
                    dst_hbm.at[pl.ds(base + (j + RING) * B, B)],
                    didx[b],
                    lsems[b],
                )
                pltpu.async_copy(
                    h_hbm.at[sidx.at[pl.ds((j + RING) * B, B)]],
                    rows[b],
                    gsems[b],
                )

        return _

    lax.fori_loop(0, NCH // RING, super_iter, None)
    for b in range(RING):
        pltpu.make_async_copy(rows[b], acc.at[didx[b]], ssems[b]).wait()
        pltpu.make_async_copy(ones_v, deg_acc.at[didx[b]], dsems[b]).wait()

    pltpu.sync_copy(dst_hbm.at[pl.ds(base + NCH * B, TAIL)], tidx)
    pltpu.async_copy(
        h_hbm.at[sidx.at[pl.ds(NCH * B, TAIL)]], trows, gsems[0]
    ).wait()
    pltpu.sync_copy(trows, acc.at[tidx], add=True)
    pltpu.sync_copy(ones_v.at[pl.ds(0, TAIL)], deg_acc.at[tidx], add=True)

    plsc.subcore_barrier()

    pltpu.sync_copy(
        acc.at[pl.ds(s * DEG_SLICE, DEG_SLICE)],
        out_hbm.at[pl.ds(c * NDEG + s * DEG_SLICE, DEG_SLICE)],
    )

    pltpu.sync_copy(
        deg_acc.at[pl.ds(s * DEG_SLICE, DEG_SLICE)],
        deg_hbm.at[pl.ds(c * NDEG + s * DEG_SLICE, DEG_SLICE)],
    )


def _finalize_body(deg_ref, parts_ref, b_ref, o_ref):
    norm = lax.rsqrt(jnp.maximum(deg_ref[0] + deg_ref[1], 1.0))
    o_ref[...] = (parts_ref[0] + parts_ref[1]) * norm + b_ref[...]


def _finalize(deg_col, parts, bias_row):
    return pl.pallas_call(
        _finalize_body,
        grid=(N // 2000,),
        in_specs=[
            pl.BlockSpec((NC, 2000, 1), lambda i: (0, i, 0)),
            pl.BlockSpec((NC, 2000, D), lambda i: (0, i, 0)),
            pl.BlockSpec((1, D), lambda i: (0, 0)),
        ],
        out_specs=pl.BlockSpec((2000, D), lambda i: (i, 0)),
        out_shape=jax.ShapeDtypeStruct((N, D), jnp.float32),
    )(deg_col, parts, bias_row)


def kernel(x, edge_index, W, b):
    src = edge_index[0].astype(jnp.int32)
    dst = edge_index[1].astype(jnp.int32)
    z1 = jnp.zeros((DEG_SLICE,), jnp.float32)
    z2 = jnp.zeros((DEG_SLICE, D), jnp.float32)
    deg_out_parts = _degree_kernel(src, z1)
    h = _scaled_matmul(deg_out_parts.reshape(NC, NDEG, 1), x, W)
    parts, deg_in_parts = _message_kernel(h, src, dst, z1, z2)
    return _finalize(
        deg_in_parts.reshape(NC, NDEG, 1),
        parts.reshape(NC, NDEG, D),
        b.reshape(1, D),
    )


# speedup vs baseline: 12.9381x; 1.1234x over previous
import functools

import jax
import jax.numpy as jnp
from jax import lax
from jax.experimental import pallas as pl
from jax.experimental.pallas import tpu as pltpu
from jax.experimental.pallas import tpu_sc as plsc

N = 10000
E = 320000
D = 128
NC = 2
NS = 16
B = 128

NPAD = 10240
EPAD = 327680
EW = EPAD // (NC * NS)
NCH = EW // B
RING = 2
NCH_DEG = EPAD // NS // B
ROWS_PER_TILE = NPAD // NS

_MESH = plsc.VectorSubcoreMesh(
    core_axis_name="c", subcore_axis_name="s", num_cores=NC, num_subcores=NS
)


@functools.partial(
    pl.kernel,
    out_type=jax.ShapeDtypeStruct((NC, NPAD), jnp.float32),
    mesh=_MESH,
    scratch_types=[
        [pltpu.VMEM((B,), jnp.int32)] * 2,
        pltpu.VMEM((B,), jnp.float32),
        pltpu.VMEM((ROWS_PER_TILE,), jnp.float32),
        pltpu.VMEM_SHARED((NPAD,), jnp.float32),
        [pltpu.SemaphoreType.DMA] * 2,
        [pltpu.SemaphoreType.DMA] * 2,
    ],
)
def _degree_kernel(edges_hbm, deg_hbm, idx, ones_v, zrow_v, deg_acc, lsems, ssems):
    c = lax.axis_index("c")
    s = lax.axis_index("s")
    one = jnp.ones((16,), jnp.float32)
    zero = jnp.zeros((16,), jnp.float32)
    for k in range(B // 16):
        ones_v[pl.ds(k * 16, 16)] = one
    for k in range(ROWS_PER_TILE // 16):
        zrow_v[pl.ds(k * 16, 16)] = zero
    pltpu.sync_copy(zrow_v, deg_acc.at[pl.ds(s * ROWS_PER_TILE, ROWS_PER_TILE)])
    plsc.subcore_barrier()

    base = (s * NC + c) * NCH * B
    for b in range(2):
        pltpu.async_copy(
            edges_hbm.at[0, pl.ds(base + b * B, B)], idx[b], lsems[b]
        )

    def pair(jj, _):
        j0 = 2 * jj
        for b in range(2):
            j = j0 + b
            pltpu.make_async_copy(
                edges_hbm.at[0, pl.ds(base, B)], idx[b], lsems[b]
            ).wait()
            pltpu.async_copy(ones_v, deg_acc.at[idx[b]], ssems[b], add=True)

            @pl.when(j + 2 < NCH)
            def _next():
                pltpu.make_async_copy(
                    ones_v, deg_acc.at[idx[b]], ssems[b]
                ).wait()
                pltpu.async_copy(
                    edges_hbm.at[0, pl.ds(base + (j + 2) * B, B)],
                    idx[b],
                    lsems[b],
                )

        return _

    lax.fori_loop(0, NCH // 2, pair, None)
    for b in range(2):
        pltpu.make_async_copy(ones_v, deg_acc.at[idx[b]], ssems[b]).wait()
    plsc.subcore_barrier()
    pltpu.sync_copy(
        deg_acc.at[pl.ds(s * ROWS_PER_TILE, ROWS_PER_TILE)],
        deg_hbm.at[c, pl.ds(s * ROWS_PER_TILE, ROWS_PER_TILE)],
    )


def _matmul_body(deg_ref, x_ref, w_ref, o_ref):
    norm = lax.rsqrt(jnp.maximum(deg_ref[0] + deg_ref[1], 1.0))
    o_ref[...] = jnp.dot(
        x_ref[...] * norm, w_ref[...], preferred_element_type=jnp.float32
    )


def _scaled_matmul(deg_col, x, w):
    return pl.pallas_call(
        _matmul_body,
        grid=(N // 2000,),
        in_specs=[
            pl.BlockSpec((NC, 2000, 1), lambda i: (0, i, 0)),
            pl.BlockSpec((2000, D), lambda i: (i, 0)),
            pl.BlockSpec((D, D), lambda i: (0, 0)),
        ],
        out_specs=pl.BlockSpec((2000, D), lambda i: (i, 0)),
        out_shape=jax.ShapeDtypeStruct((NPAD, D), jnp.float32),
    )(deg_col, x, w)


@functools.partial(
    pl.kernel,
    out_type=(
        jax.ShapeDtypeStruct((NC, NPAD, D), jnp.float32),
        jax.ShapeDtypeStruct((NC, NPAD), jnp.float32),
    ),
    mesh=_MESH,
    scratch_types=[
        pltpu.VMEM((EW,), jnp.int32),
        [pltpu.VMEM((B,), jnp.int32)] * RING,
        [pltpu.VMEM((B, D), jnp.float32)] * RING,
        pltpu.VMEM((16, D), jnp.float32),
        pltpu.VMEM((B,), jnp.float32),
        pltpu.VMEM((ROWS_PER_TILE,), jnp.float32),
        pltpu.VMEM_SHARED((NPAD, D), jnp.float32),
        pltpu.VMEM_SHARED((NPAD,), jnp.float32),
        [pltpu.SemaphoreType.DMA] * RING,
        [pltpu.SemaphoreType.DMA] * RING,
        [pltpu.SemaphoreType.DMA] * RING,
        [pltpu.SemaphoreType.DMA] * RING,
    ],
)
def _message_kernel(
    h_hbm, edges_hbm, out_hbm, deg_hbm,
    sidx, didx, rows, zbuf, ones_v, zrow_v, acc, deg_acc,
    gsems, ssems, lsems, dsems,
):
    c = lax.axis_index("c")
    s = lax.axis_index("s")
    wid = s * NC + c
    base = wid * EW

    one = jnp.ones((16,), jnp.float32)
    zero = jnp.zeros((16,), jnp.float32)
    for k in range(B // 16):
        ones_v[pl.ds(k * 16, 16)] = one
    for k in range(ROWS_PER_TILE // 16):
        zrow_v[pl.ds(k * 16, 16)] = zero

    def zrow_body(r, _):
        for k in range(D // 16):
            zbuf[r, pl.ds(k * 16, 16)] = zero
        return _

    lax.fori_loop(0, 16, zrow_body, None)

    def zcopy(k, _):
        pltpu.sync_copy(zbuf, acc.at[pl.ds(s * ROWS_PER_TILE + k * 16, 16)])
        return _

    lax.fori_loop(0, ROWS_PER_TILE // 16, zcopy, None)
    pltpu.sync_copy(zrow_v, deg_acc.at[pl.ds(s * ROWS_PER_TILE, ROWS_PER_TILE)])
    plsc.subcore_barrier()

    pltpu.sync_copy(edges_hbm.at[0, pl.ds(base, EW)], sidx)

    for b in range(RING):
        pltpu.async_copy(
            edges_hbm.at[1, pl.ds(base + b * B, B)], didx[b], lsems[b]
        )
        pltpu.async_copy(
            h_hbm.at[sidx.at[pl.ds(b * B, B)]], rows[b], gsems[b]
        )

    def super_iter(jj, _):
        j0 = jj * RING
        for b in range(RING):
            j = j0 + b
            pltpu.make_async_copy(
                h_hbm.at[sidx.at[pl.ds(0, B)]], rows[b], gsems[b]
            ).wait()
            pltpu.make_async_copy(
                edges_hbm.at[1, pl.ds(base, B)], didx[b], lsems[b]
            ).wait()
            pltpu.async_copy(rows[b], acc.at[didx[b]], ssems[b], add=True)
            pltpu.async_copy(ones_v, deg_acc.at[didx[b]], dsems[b], add=True)

            @pl.when(j + RING < NCH)
            def _next():
                pltpu.make_async_copy(
                    rows[b], acc.at[didx[b]], ssems[b]
                ).wait()
                pltpu.make_async_copy(
                    ones_v, deg_acc.at[didx[b]], dsems[b]
                ).wait()
                pltpu.async_copy(
                    edges_hbm.at[1, pl.ds(base + (j + RING) * B, B)],
                    didx[b],
                    lsems[b],
                )
                pltpu.async_copy(
                    h_hbm.at[sidx.at[pl.ds((j + RING) * B, B)]],
                    rows[b],
                    gsems[b],
                )

        return _

    lax.fori_loop(0, NCH // RING, super_iter, None)
    for b in range(RING):
        pltpu.make_async_copy(rows[b], acc.at[didx[b]], ssems[b]).wait()
        pltpu.make_async_copy(ones_v, deg_acc.at[didx[b]], dsems[b]).wait()
    plsc.subcore_barrier()
    pltpu.sync_copy(
        acc.at[pl.ds(s * ROWS_PER_TILE, ROWS_PER_TILE)],
        out_hbm.at[c, pl.ds(s * ROWS_PER_TILE, ROWS_PER_TILE)],
    )
    pltpu.sync_copy(
        deg_acc.at[pl.ds(s * ROWS_PER_TILE, ROWS_PER_TILE)],
        deg_hbm.at[c, pl.ds(s * ROWS_PER_TILE, ROWS_PER_TILE)],
    )


def _finalize_body(deg_ref, parts_ref, b_ref, o_ref):
    norm = lax.rsqrt(jnp.maximum(deg_ref[0] + deg_ref[1], 1.0))
    o_ref[...] = (parts_ref[0] + parts_ref[1]) * norm + b_ref[...]


def _finalize(deg_col, parts, bias_row):
    return pl.pallas_call(
        _finalize_body,
        grid=(N // 2000,),
        in_specs=[
            pl.BlockSpec((NC, 2000, 1), lambda i: (0, i, 0)),
            pl.BlockSpec((NC, 2000, D), lambda i: (0, i, 0)),
            pl.BlockSpec((1, D), lambda i: (0, 0)),
        ],
        out_specs=pl.BlockSpec((2000, D), lambda i: (i, 0)),
        out_shape=jax.ShapeDtypeStruct((N, D), jnp.float32),
    )(deg_col, parts, bias_row)


def kernel(x, edge_index, W, b):
    pad = N + (jnp.arange(EPAD - E, dtype=jnp.int32) % (NPAD - N))
    edges2 = jnp.concatenate(
        [edge_index.astype(jnp.int32), jnp.broadcast_to(pad, (2, EPAD - E))],
        axis=1,
    )

    deg_out_parts = _degree_kernel(edges2)
    h = _scaled_matmul(deg_out_parts.reshape(NC, NPAD, 1), x, W)
    parts, deg_in_parts = _message_kernel(h, edges2)
    return _finalize(
        deg_in_parts.reshape(NC, NPAD, 1), parts, b.reshape(1, D)
    )
